# baseline (speedup 1.0000x reference)
import functools
import math

import jax
import jax.numpy as jnp
from jax import lax
from jax.experimental import pallas as pl
from jax.experimental.pallas import tpu as pltpu
from jax.experimental.pallas import tpu_sc as plsc

_DM = 256
_NH = 4
_DH = 64
_NL = 6
_B = 16
_VL = 64
_NT = 50
_DV = 1408
_NSEL = 8

_NG = 4
_BG = _B // _NG
_TG = _BG * _VL + _BG

_LANES = 16
_NCH = _VL // _LANES

_LYR_KEYS = ("Wq", "bq", "Wk", "bk", "Wv", "bv", "Wo", "bo",
             "W1", "b1", "W2", "b2", "ln1_g", "ln1_b", "ln2_g", "ln2_b")


def _dot(a, b):
    return lax.dot_general(a, b, (((1,), (0,)), ((), ())),
                           preferred_element_type=jnp.float32)


def _dot_t(a, b):
    return lax.dot_general(a, b, (((1,), (1,)), ((), ())),
                           preferred_element_type=jnp.float32)


def _ln(x, g, b):
    m = jnp.mean(x, axis=1, keepdims=True)
    v = jnp.mean((x - m) ** 2, axis=1, keepdims=True)
    return (x - m) / jnp.sqrt(v + 1e-5) * g + b


def _enc_body(*refs):
    (xcls_ref, xtxt_ref, wv_ref, bv_ref, wt_ref, bt_ref, we_ref, be_ref,
     mod0_ref, mod3_ref) = refs[:10]
    lrefs = refs[10:10 + 16 * _NL]
    wl_ref = refs[10 + 16 * _NL]
    val_ref = refs[11 + 16 * _NL]

    nvis = _BG * _VL
    v = _dot(xcls_ref[...], wv_ref[...]) + bv_ref[...]
    t = _dot(xtxt_ref[0], wt_ref[...]) + bt_ref[...]
    ve = _dot(v, we_ref[...]) + be_ref[...]
    te = _dot(t, we_ref[...]) + be_ref[...]
    x = jnp.concatenate([ve, te], axis=0) * math.sqrt(_DM)
    r = lax.broadcasted_iota(jnp.int32, (_TG, 1), 0)
    is_txt = r >= nvis
    x = x + jnp.where(is_txt, mod0_ref[...], mod3_ref[...])

    rc = lax.broadcasted_iota(jnp.int32, (1, _TG), 1)
    blk_r = jnp.where(r < nvis, r // _VL, r - nvis)
    blk_c = jnp.where(rc < nvis, rc // _VL, rc - nvis)
    bias = jnp.where(blk_r == blk_c, 0.0, -1e30)

    for i in range(_NL):
        (wq, bq, wk, bk, wvv, bvv, wo, bo, w1, b1, w2, b2,
         g1, c1, g2, c2) = lrefs[16 * i:16 * (i + 1)]
        q = _dot(x, wq[...]) + bq[...]
        k = _dot(x, wk[...]) + bk[...]
        vv = _dot(x, wvv[...]) + bvv[...]
        heads = []
        for h in range(_NH):
            sl = slice(h * _DH, (h + 1) * _DH)
            s = _dot_t(q[:, sl], k[:, sl]) * (1.0 / math.sqrt(_DH)) + bias
            m = jnp.max(s, axis=1, keepdims=True)
            e = jnp.exp(s - m)
            a = e / jnp.sum(e, axis=1, keepdims=True)
            heads.append(_dot(a, vv[:, sl]))
        attn = _dot(jnp.concatenate(heads, axis=1), wo[...]) + bo[...]
        x = _ln(x + attn, g1[...], c1[...])
        ff = _dot(jnp.maximum(_dot(x, w1[...]) + b1[...], 0.0),
                  w2[...]) + b2[...]
        x = _ln(x + ff, g2[...], c2[...])

    w = wl_ref[...]
    vals = jnp.concatenate(
        [_dot_t(w, x[n * _VL:(n + 1) * _VL]) for n in range(_BG)], axis=0)
    val_ref[0] = jnp.concatenate(
        [vals, jnp.zeros((_BG, 128 - _VL), jnp.float32)], axis=1)


def _full(shape):
    return pl.BlockSpec(shape, lambda g: (0,) * len(shape))


def _encode(xcls, xtxt3, *weights):
    return pl.pallas_call(
        _enc_body,
        grid=(_NG,),
        in_specs=[
            pl.BlockSpec((_BG * _VL, _DV), lambda g: (g, 0)),
            pl.BlockSpec((1, _BG, 2048), lambda g: (g, 0, 0)),
        ] + [_full(w.shape) for w in weights],
        out_specs=pl.BlockSpec((1, _BG, 128), lambda g: (g, 0, 0)),
        out_shape=jax.ShapeDtypeStruct((_NG, _BG, 128), jnp.float32),
    )(xcls, xtxt3, *weights)


def _vgather(x, idx):
    dn = lax.GatherDimensionNumbers(offset_dims=(), collapsed_slice_dims=(0,),
                                    start_index_map=(0,))
    return lax.gather(x, idx[:, None], dn, (1,),
                      mode=lax.GatherScatterMode.PROMISE_IN_BOUNDS)


def _bfly(x, op, i):
    for k in (1, 2, 4, 8):
        x = op(x, _vgather(x, jnp.bitwise_xor(i, k)))
    return x


def _topk_body(vals_ref, idx_ref, row_v, out_v):
    core = lax.axis_index("c")
    row = lax.axis_index("s")

    @pl.when(core == 0)
    def _():
        pltpu.sync_copy(vals_ref.at[row], row_v)
        i = lax.iota(jnp.int32, _LANES)
        chunks = [row_v[pl.ds(c * _LANES, _LANES)] for c in range(_NCH)]
        lanes = [i + c * _LANES for c in range(_NCH)]
        ninf = jnp.float32(-jnp.inf)
        sel = [jnp.zeros((_LANES,), jnp.bool_) for _ in range(_NCH)]
        for _s in range(_NSEL):
            masked = [jnp.where(sel[c], ninf, chunks[c]) for c in range(_NCH)]
            m = _bfly(masked[0], jnp.maximum, i)
            for c in range(1, _NCH):
                m = jnp.maximum(m, _bfly(masked[c], jnp.maximum, i))
            amax = jnp.full((_LANES,), _VL, jnp.int32)
            for c in range(_NCH):
                cand = jnp.where(masked[c] == m, lanes[c], _VL)
                amax = jnp.minimum(amax, _bfly(cand, jnp.minimum, i))
            for c in range(_NCH):
                sel[c] = jnp.logical_or(sel[c], lanes[c] == amax)
        out = jnp.zeros((_LANES,), jnp.int32)
        for s in range(_NSEL):
            mn = jnp.full((_LANES,), _VL, jnp.int32)
            for c in range(_NCH):
                cand = jnp.where(sel[c], lanes[c], _VL)
                mn = jnp.minimum(mn, _bfly(cand, jnp.minimum, i))
            out = jnp.where(i == s, mn, out)
            for c in range(_NCH):
                sel[c] = jnp.logical_and(sel[c], lanes[c] != mn)
        out_v[pl.ds(0, _LANES)] = out
        pltpu.sync_copy(out_v, idx_ref.at[row])


@functools.cache
def _sc_topk():
    return pl.kernel(
        _topk_body,
        out_type=jax.ShapeDtypeStruct((_B, 128), jnp.int32),
        mesh=plsc.VectorSubcoreMesh(core_axis_name="c", subcore_axis_name="s"),
        scratch_types=[
            pltpu.VMEM((128,), jnp.float32),
            pltpu.VMEM((128,), jnp.int32),
        ],
    )


def _gather_body(idx_ref, src_ref, out_ref):
    del idx_ref
    out_ref[...] = src_ref[...]


def _gather(x_vis, idx_flat):
    grid_spec = pltpu.PrefetchScalarGridSpec(
        num_scalar_prefetch=1,
        grid=(_B * _NSEL,),
        in_specs=[pl.BlockSpec((1, 1, _NT, _DV),
                               lambda g, idx: (g // _NSEL, idx[g], 0, 0))],
        out_specs=pl.BlockSpec((1, 1, _NT, _DV),
                               lambda g, idx: (g // _NSEL, g % _NSEL, 0, 0)),
    )
    return pl.pallas_call(
        _gather_body,
        grid_spec=grid_spec,
        out_shape=jax.ShapeDtypeStruct((_B, _NSEL, _NT, _DV), jnp.float32),
    )(idx_flat, x_vis)


def kernel(x_vis, x_txt, params):
    p = params
    r2 = lambda a: a.reshape(1, -1)

    xcls = x_vis[:, :, 0, :].reshape(_B * _VL, _DV)
    xtxt3 = x_txt.reshape(_NG, _BG, -1)

    lweights = []
    for l in p["layers"]:
        for kname in _LYR_KEYS:
            a = l[kname]
            lweights.append(a if a.ndim == 2 else r2(a))

    vals = _encode(
        xcls, xtxt3,
        p["v_emb"]["W"], r2(p["v_emb"]["b"]),
        p["t_emb"]["W"], r2(p["t_emb"]["b"]),
        p["emb"]["W"], r2(p["emb"]["b"]),
        r2(p["mod_table"][0]), r2(p["mod_table"][3]),
        *lweights,
        p["logits"]["W"].reshape(1, -1),
    ).reshape(_B, 128)

    idx = _sc_topk()(vals)
    return _gather(x_vis, idx[:, :_NSEL].reshape(-1))


# speedup vs baseline: 4.0377x; 4.0377x over previous
import functools
import math

import jax
import jax.numpy as jnp
from jax import lax
from jax.experimental import pallas as pl
from jax.experimental.pallas import tpu as pltpu
from jax.experimental.pallas import tpu_sc as plsc

_DM = 256
_NH = 4
_DH = 64
_NL = 6
_B = 16
_VL = 64
_NT = 50
_DV = 1408
_NSEL = 8

_NG = 4
_BG = _B // _NG
_TG = _BG * _VL + _BG

_LANES = 16
_NCH = _VL // _LANES

_LYR_KEYS = ("Wq", "bq", "Wk", "bk", "Wv", "bv", "Wo", "bo",
             "W1", "b1", "W2", "b2", "ln1_g", "ln1_b", "ln2_g", "ln2_b")


def _dot(a, b):
    return lax.dot_general(a, b, (((1,), (0,)), ((), ())),
                           preferred_element_type=jnp.float32)


def _dot_t(a, b):
    return lax.dot_general(a, b, (((1,), (1,)), ((), ())),
                           preferred_element_type=jnp.float32)


def _ln(x, g, b):
    m = jnp.mean(x, axis=1, keepdims=True)
    v = jnp.mean((x - m) ** 2, axis=1, keepdims=True)
    return (x - m) / jnp.sqrt(v + 1e-5) * g + b


def _enc_body(*refs):
    (xcls_ref, xtxt_ref, wv_ref, bv_ref, wt_ref, bt_ref, we_ref, be_ref,
     mod0_ref, mod3_ref) = refs[:10]
    lrefs = refs[10:10 + 16 * _NL]
    wl_ref = refs[10 + 16 * _NL]
    val_ref = refs[11 + 16 * _NL]

    nvis = _BG * _VL
    v = _dot(xcls_ref[...], wv_ref[...]) + bv_ref[...]
    t = _dot(xtxt_ref[0], wt_ref[...]) + bt_ref[...]
    ve = _dot(v, we_ref[...]) + be_ref[...]
    te = _dot(t, we_ref[...]) + be_ref[...]
    x = jnp.concatenate([ve, te], axis=0) * math.sqrt(_DM)
    r = lax.broadcasted_iota(jnp.int32, (_TG, 1), 0)
    is_txt = r >= nvis
    x = x + jnp.where(is_txt, mod0_ref[...], mod3_ref[...])

    rc = lax.broadcasted_iota(jnp.int32, (1, _TG), 1)
    blk_r = jnp.where(r < nvis, r // _VL, r - nvis)
    blk_c = jnp.where(rc < nvis, rc // _VL, rc - nvis)
    bias = jnp.where(blk_r == blk_c, 0.0, -1e30)

    for i in range(_NL):
        (wq, bq, wk, bk, wvv, bvv, wo, bo, w1, b1, w2, b2,
         g1, c1, g2, c2) = lrefs[16 * i:16 * (i + 1)]
        q = _dot(x, wq[...]) + bq[...]
        k = _dot(x, wk[...]) + bk[...]
        vv = _dot(x, wvv[...]) + bvv[...]
        heads = []
        for h in range(_NH):
            sl = slice(h * _DH, (h + 1) * _DH)
            s = _dot_t(q[:, sl], k[:, sl]) * (1.0 / math.sqrt(_DH)) + bias
            m = jnp.max(s, axis=1, keepdims=True)
            e = jnp.exp(s - m)
            a = e / jnp.sum(e, axis=1, keepdims=True)
            heads.append(_dot(a, vv[:, sl]))
        attn = _dot(jnp.concatenate(heads, axis=1), wo[...]) + bo[...]
        x = _ln(x + attn, g1[...], c1[...])
        ff = _dot(jnp.maximum(_dot(x, w1[...]) + b1[...], 0.0),
                  w2[...]) + b2[...]
        x = _ln(x + ff, g2[...], c2[...])

    w = wl_ref[...]
    vals = jnp.concatenate(
        [_dot_t(w, x[n * _VL:(n + 1) * _VL]) for n in range(_BG)], axis=0)
    val_ref[0] = jnp.concatenate(
        [vals, jnp.zeros((_BG, 128 - _VL), jnp.float32)], axis=1)


def _full(shape):
    return pl.BlockSpec(shape, lambda g: (0,) * len(shape))


def _encode(xcls, xtxt3, *weights):
    return pl.pallas_call(
        _enc_body,
        grid=(_NG,),
        in_specs=[
            pl.BlockSpec((_BG * _VL, _DV), lambda g: (g, 0)),
            pl.BlockSpec((1, _BG, 2048), lambda g: (g, 0, 0)),
        ] + [_full(w.shape) for w in weights],
        out_specs=pl.BlockSpec((1, _BG, 128), lambda g: (g, 0, 0)),
        out_shape=jax.ShapeDtypeStruct((_NG, _BG, 128), jnp.float32),
    )(xcls, xtxt3, *weights)


def _vgather(x, idx):
    dn = lax.GatherDimensionNumbers(offset_dims=(), collapsed_slice_dims=(0,),
                                    start_index_map=(0,))
    return lax.gather(x, idx[:, None], dn, (1,),
                      mode=lax.GatherScatterMode.PROMISE_IN_BOUNDS)


def _bfly(x, op, i):
    for k in (1, 2, 4, 8):
        x = op(x, _vgather(x, jnp.bitwise_xor(i, k)))
    return x


_PAIRS_PW = _B * _NSEL // 32
_ROWS_PW = _PAIRS_PW * _NT
_CHUNKS = (_ROWS_PW + _LANES - 1) // _LANES


def _topk_row(row_v, i):
    chunks = [row_v[pl.ds(c * _LANES, _LANES)] for c in range(_NCH)]
    lanes = [i + c * _LANES for c in range(_NCH)]
    ninf = jnp.float32(-jnp.inf)
    sel = [jnp.zeros((_LANES,), jnp.bool_) for _ in range(_NCH)]
    for _s in range(_NSEL):
        masked = [jnp.where(sel[c], ninf, chunks[c]) for c in range(_NCH)]
        m = _bfly(masked[0], jnp.maximum, i)
        for c in range(1, _NCH):
            m = jnp.maximum(m, _bfly(masked[c], jnp.maximum, i))
        amax = jnp.full((_LANES,), _VL, jnp.int32)
        for c in range(_NCH):
            cand = jnp.where(masked[c] == m, lanes[c], _VL)
            amax = jnp.minimum(amax, _bfly(cand, jnp.minimum, i))
        for c in range(_NCH):
            sel[c] = jnp.logical_or(sel[c], lanes[c] == amax)
    out = jnp.zeros((_LANES,), jnp.int32)
    for s in range(_NSEL):
        mn = jnp.full((_LANES,), _VL, jnp.int32)
        for c in range(_NCH):
            cand = jnp.where(sel[c], lanes[c], _VL)
            mn = jnp.minimum(mn, _bfly(cand, jnp.minimum, i))
        out = jnp.where(i == s, mn, out)
        for c in range(_NCH):
            sel[c] = jnp.logical_and(sel[c], lanes[c] != mn)
    return out


def _select_body(vals_ref, table_ref, out_ref, row_v, j_vm, gidx_vm, sidx_vm,
                 buf_a, buf_b, gsem_a, gsem_b, wsem_a, wsem_b):
    wid = lax.axis_index("s") * 2 + lax.axis_index("c")
    b = lax.shift_right_logical(wid, 1)
    s0 = (wid & 1) * _PAIRS_PW
    i = lax.iota(jnp.int32, _LANES)

    pltpu.sync_copy(vals_ref.at[b], row_v)
    j_vm[...] = _topk_row(row_v, i)

    for c in range(_CHUNKS):
        r = jnp.minimum(i + c * _LANES, _ROWS_PW - 1)
        p = (jnp.where(r >= _NT, 1, 0) + jnp.where(r >= 2 * _NT, 1, 0)
             + jnp.where(r >= 3 * _NT, 1, 0))
        s = s0 + p
        t = r - p * _NT
        j = _vgather(j_vm[...], s)
        gidx_vm[c] = (b * _NT + t) * _VL + j
        sidx_vm[c] = (b * _NT + t) * _NSEL + s

    bufs = (buf_a, buf_b)
    gsems = (gsem_a, gsem_b)
    wsems = (wsem_a, wsem_b)
    gh = [None, None]
    wh = [None, None]
    for c in range(_CHUNKS):
        k = c % 2
        if wh[k] is not None:
            wh[k].wait()
        gh[k] = pltpu.async_copy(table_ref.at[gidx_vm.at[c]], bufs[k],
                                 gsems[k])
        gh[k].wait()
        wh[k] = pltpu.async_copy(bufs[k], out_ref.at[sidx_vm.at[c]], wsems[k])
    for k in range(2):
        if wh[k] is not None:
            wh[k].wait()


@functools.cache
def _sc_select():
    return pl.kernel(
        _select_body,
        out_type=jax.ShapeDtypeStruct((_B * _NT * _NSEL, _DV), jnp.float32),
        mesh=plsc.VectorSubcoreMesh(core_axis_name="c", subcore_axis_name="s"),
        scratch_types=[
            pltpu.VMEM((128,), jnp.float32),
            pltpu.VMEM((_LANES,), jnp.int32),
            pltpu.VMEM((_CHUNKS, _LANES), jnp.int32),
            pltpu.VMEM((_CHUNKS, _LANES), jnp.int32),
            pltpu.VMEM((_LANES, _DV), jnp.float32),
            pltpu.VMEM((_LANES, _DV), jnp.float32),
            pltpu.SemaphoreType.DMA,
            pltpu.SemaphoreType.DMA,
            pltpu.SemaphoreType.DMA,
            pltpu.SemaphoreType.DMA,
        ],
    )


def kernel(x_vis, x_txt, params):
    p = params
    r2 = lambda a: a.reshape(1, -1)

    xcls = x_vis[:, :, 0, :].reshape(_B * _VL, _DV)
    xtxt3 = x_txt.reshape(_NG, _BG, -1)

    lweights = []
    for l in p["layers"]:
        for kname in _LYR_KEYS:
            a = l[kname]
            lweights.append(a if a.ndim == 2 else r2(a))

    vals = _encode(
        xcls, xtxt3,
        p["v_emb"]["W"], r2(p["v_emb"]["b"]),
        p["t_emb"]["W"], r2(p["t_emb"]["b"]),
        p["emb"]["W"], r2(p["emb"]["b"]),
        r2(p["mod_table"][0]), r2(p["mod_table"][3]),
        *lweights,
        p["logits"]["W"].reshape(1, -1),
    ).reshape(_B, 128)

    table = jnp.transpose(x_vis, (0, 2, 1, 3)).reshape(_B * _NT * _VL, _DV)
    out = _sc_select()(vals, table)
    return jnp.transpose(out.reshape(_B, _NT, _NSEL, _DV), (0, 2, 1, 3))


# speedup vs baseline: 4.5277x; 1.1214x over previous
import functools
import math

import jax
import jax.numpy as jnp
from jax import lax
from jax.experimental import pallas as pl
from jax.experimental.pallas import tpu as pltpu
from jax.experimental.pallas import tpu_sc as plsc

_DM = 256
_NH = 4
_DH = 64
_NL = 6
_B = 16
_VL = 64
_NT = 50
_DV = 1408
_NSEL = 8

_NG = 2
_BG = _B // _NG
_TG = _BG * _VL + _BG

_LANES = 16
_NCH = _VL // _LANES

_LYR_KEYS = ("Wq", "bq", "Wk", "bk", "Wv", "bv", "Wo", "bo",
             "W1", "b1", "W2", "b2", "ln1_g", "ln1_b", "ln2_g", "ln2_b")


def _dot(a, b):
    return lax.dot_general(a, b, (((1,), (0,)), ((), ())),
                           preferred_element_type=jnp.float32)


def _dot_t(a, b):
    return lax.dot_general(a, b, (((1,), (1,)), ((), ())),
                           preferred_element_type=jnp.float32)


def _ln(x, g, b):
    m = jnp.mean(x, axis=1, keepdims=True)
    v = jnp.mean((x - m) ** 2, axis=1, keepdims=True)
    return (x - m) / jnp.sqrt(v + 1e-5) * g + b


def _enc_body(*refs):
    (xcls_ref, xtxt_ref, wv_ref, bv_ref, wt_ref, bt_ref, we_ref, be_ref,
     mod0_ref, mod3_ref) = refs[:10]
    lrefs = refs[10:10 + 16 * _NL]
    wl_ref = refs[10 + 16 * _NL]
    val_ref = refs[11 + 16 * _NL]

    nvis = _BG * _VL
    v = _dot(xcls_ref[...], wv_ref[...]) + bv_ref[...]
    t = _dot(xtxt_ref[0], wt_ref[...]) + bt_ref[...]
    ve = _dot(v, we_ref[...]) + be_ref[...]
    te = _dot(t, we_ref[...]) + be_ref[...]
    x = jnp.concatenate([ve, te], axis=0) * math.sqrt(_DM)
    r = lax.broadcasted_iota(jnp.int32, (_TG, 1), 0)
    is_txt = r >= nvis
    x = x + jnp.where(is_txt, mod0_ref[...], mod3_ref[...])

    rc = lax.broadcasted_iota(jnp.int32, (1, _TG), 1)
    blk_r = jnp.where(r < nvis, r // _VL, r - nvis)
    blk_c = jnp.where(rc < nvis, rc // _VL, rc - nvis)
    bias = jnp.where(blk_r == blk_c, 0.0, -1e30)

    for i in range(_NL):
        (wq, bq, wk, bk, wvv, bvv, wo, bo, w1, b1, w2, b2,
         g1, c1, g2, c2) = lrefs[16 * i:16 * (i + 1)]
        q = _dot(x, wq[...]) + bq[...]
        k = _dot(x, wk[...]) + bk[...]
        vv = _dot(x, wvv[...]) + bvv[...]
        heads = []
        for h in range(_NH):
            sl = slice(h * _DH, (h + 1) * _DH)
            s = _dot_t(q[:, sl], k[:, sl]) * (1.0 / math.sqrt(_DH)) + bias
            e = jnp.exp(s)
            a = e / jnp.sum(e, axis=1, keepdims=True)
            heads.append(_dot(a, vv[:, sl]))
        attn = _dot(jnp.concatenate(heads, axis=1), wo[...]) + bo[...]
        x = _ln(x + attn, g1[...], c1[...])
        ff = _dot(jnp.maximum(_dot(x, w1[...]) + b1[...], 0.0),
                  w2[...]) + b2[...]
        x = _ln(x + ff, g2[...], c2[...])

    w = wl_ref[...]
    vals = jnp.concatenate(
        [_dot_t(w, x[n * _VL:(n + 1) * _VL]) for n in range(_BG)], axis=0)
    val_ref[0] = jnp.concatenate(
        [vals, jnp.zeros((_BG, 128 - _VL), jnp.float32)], axis=1)


def _full(shape):
    return pl.BlockSpec(shape, lambda g: (0,) * len(shape))


def _encode(xcls, xtxt3, *weights):
    return pl.pallas_call(
        _enc_body,
        grid=(_NG,),
        in_specs=[
            pl.BlockSpec((_BG * _VL, _DV), lambda g: (g, 0)),
            pl.BlockSpec((1, _BG, 2048), lambda g: (g, 0, 0)),
        ] + [_full(w.shape) for w in weights],
        out_specs=pl.BlockSpec((1, _BG, 128), lambda g: (g, 0, 0)),
        out_shape=jax.ShapeDtypeStruct((_NG, _BG, 128), jnp.float32),
    )(xcls, xtxt3, *weights)


def _vgather(x, idx):
    dn = lax.GatherDimensionNumbers(offset_dims=(), collapsed_slice_dims=(0,),
                                    start_index_map=(0,))
    return lax.gather(x, idx[:, None], dn, (1,),
                      mode=lax.GatherScatterMode.PROMISE_IN_BOUNDS)


def _bfly(x, op, i):
    for k in (1, 2, 4, 8):
        x = op(x, _vgather(x, jnp.bitwise_xor(i, k)))
    return x


_PAIRS_PW = _B * _NSEL // 32
_ROWS_PW = _PAIRS_PW * _NT
_CHUNKS = (_ROWS_PW + _LANES - 1) // _LANES


def _topk_row(row_v, i):
    chunks = [row_v[pl.ds(c * _LANES, _LANES)] for c in range(_NCH)]
    lanes = [i + c * _LANES for c in range(_NCH)]
    ninf = jnp.float32(-jnp.inf)
    sel = [jnp.zeros((_LANES,), jnp.bool_) for _ in range(_NCH)]
    for _s in range(_NSEL):
        masked = [jnp.where(sel[c], ninf, chunks[c]) for c in range(_NCH)]
        m = _bfly(masked[0], jnp.maximum, i)
        for c in range(1, _NCH):
            m = jnp.maximum(m, _bfly(masked[c], jnp.maximum, i))
        amax = jnp.full((_LANES,), _VL, jnp.int32)
        for c in range(_NCH):
            cand = jnp.where(masked[c] == m, lanes[c], _VL)
            amax = jnp.minimum(amax, _bfly(cand, jnp.minimum, i))
        for c in range(_NCH):
            sel[c] = jnp.logical_or(sel[c], lanes[c] == amax)
    out = jnp.zeros((_LANES,), jnp.int32)
    for s in range(_NSEL):
        mn = jnp.full((_LANES,), _VL, jnp.int32)
        for c in range(_NCH):
            cand = jnp.where(sel[c], lanes[c], _VL)
            mn = jnp.minimum(mn, _bfly(cand, jnp.minimum, i))
        out = jnp.where(i == s, mn, out)
        for c in range(_NCH):
            sel[c] = jnp.logical_and(sel[c], lanes[c] != mn)
    return out


def _select_body(vals_ref, table_ref, out_ref, row_v, j_vm, gidx_vm, sidx_vm,
                 buf_a, buf_b, gsem_a, gsem_b, wsem_a, wsem_b):
    wid = lax.axis_index("s") * 2 + lax.axis_index("c")
    b = lax.shift_right_logical(wid, 1)
    s0 = (wid & 1) * _PAIRS_PW
    i = lax.iota(jnp.int32, _LANES)

    pltpu.sync_copy(vals_ref.at[b], row_v)
    j_vm[...] = _topk_row(row_v, i)

    for c in range(_CHUNKS):
        r = jnp.minimum(i + c * _LANES, _ROWS_PW - 1)
        p = (jnp.where(r >= _NT, 1, 0) + jnp.where(r >= 2 * _NT, 1, 0)
             + jnp.where(r >= 3 * _NT, 1, 0))
        s = s0 + p
        t = r - p * _NT
        j = _vgather(j_vm[...], s)
        gidx_vm[c] = (b * _NT + t) * _VL + j
        sidx_vm[c] = (b * _NT + t) * _NSEL + s

    bufs = (buf_a, buf_b)
    gsems = (gsem_a, gsem_b)
    wsems = (wsem_a, wsem_b)
    gh = [None, None]
    wh = [None, None]
    for c in range(_CHUNKS):
        k = c % 2
        if wh[k] is not None:
            wh[k].wait()
        gh[k] = pltpu.async_copy(table_ref.at[gidx_vm.at[c]], bufs[k],
                                 gsems[k])
        gh[k].wait()
        wh[k] = pltpu.async_copy(bufs[k], out_ref.at[sidx_vm.at[c]], wsems[k])
    for k in range(2):
        if wh[k] is not None:
            wh[k].wait()


@functools.cache
def _sc_select():
    return pl.kernel(
        _select_body,
        out_type=jax.ShapeDtypeStruct((_B * _NT * _NSEL, _DV), jnp.float32),
        mesh=plsc.VectorSubcoreMesh(core_axis_name="c", subcore_axis_name="s"),
        scratch_types=[
            pltpu.VMEM((128,), jnp.float32),
            pltpu.VMEM((_LANES,), jnp.int32),
            pltpu.VMEM((_CHUNKS, _LANES), jnp.int32),
            pltpu.VMEM((_CHUNKS, _LANES), jnp.int32),
            pltpu.VMEM((_LANES, _DV), jnp.float32),
            pltpu.VMEM((_LANES, _DV), jnp.float32),
            pltpu.SemaphoreType.DMA,
            pltpu.SemaphoreType.DMA,
            pltpu.SemaphoreType.DMA,
            pltpu.SemaphoreType.DMA,
        ],
    )


def kernel(x_vis, x_txt, params):
    p = params
    r2 = lambda a: a.reshape(1, -1)

    xcls = x_vis[:, :, 0, :].reshape(_B * _VL, _DV)
    xtxt3 = x_txt.reshape(_NG, _BG, -1)

    lweights = []
    for l in p["layers"]:
        for kname in _LYR_KEYS:
            a = l[kname]
            lweights.append(a if a.ndim == 2 else r2(a))

    vals = _encode(
        xcls, xtxt3,
        p["v_emb"]["W"], r2(p["v_emb"]["b"]),
        p["t_emb"]["W"], r2(p["t_emb"]["b"]),
        p["emb"]["W"], r2(p["emb"]["b"]),
        r2(p["mod_table"][0]), r2(p["mod_table"][3]),
        *lweights,
        p["logits"]["W"].reshape(1, -1),
    ).reshape(_B, 128)

    table = jnp.transpose(x_vis, (0, 2, 1, 3)).reshape(_B * _NT * _VL, _DV)
    out = _sc_select()(vals, table)
    return jnp.transpose(out.reshape(_B, _NT, _NSEL, _DV), (0, 2, 1, 3))


# speedup vs baseline: 4.7396x; 1.0468x over previous
import functools
import math

import jax
import jax.numpy as jnp
from jax import lax
from jax.experimental import pallas as pl
from jax.experimental.pallas import tpu as pltpu
from jax.experimental.pallas import tpu_sc as plsc

_DM = 256
_NH = 4
_DH = 64
_NL = 6
_B = 16
_VL = 64
_NT = 50
_DV = 1408
_NSEL = 8

_NG = 2
_BG = _B // _NG
_TG = _BG * _VL + _BG

_LANES = 16
_NCH = _VL // _LANES

_LYR_KEYS = ("Wq", "bq", "Wk", "bk", "Wv", "bv", "Wo", "bo",
             "W1", "b1", "W2", "b2", "ln1_g", "ln1_b", "ln2_g", "ln2_b")


def _dot(a, b):
    return lax.dot_general(a, b, (((1,), (0,)), ((), ())),
                           preferred_element_type=jnp.float32)


def _dot_t(a, b):
    return lax.dot_general(a, b, (((1,), (1,)), ((), ())),
                           preferred_element_type=jnp.float32)


def _ln(x, g, b):
    m = jnp.mean(x, axis=1, keepdims=True)
    v = jnp.mean((x - m) ** 2, axis=1, keepdims=True)
    return (x - m) / jnp.sqrt(v + 1e-5) * g + b


def _enc_body(*refs):
    (xcls_ref, xtxt_ref, wv_ref, bv_ref, wt_ref, bt_ref, we_ref, be_ref,
     mod0_ref, mod3_ref) = refs[:10]
    lrefs = refs[10:10 + 16 * _NL]
    wl_ref = refs[10 + 16 * _NL]
    val_ref = refs[11 + 16 * _NL]

    nvis = _BG * _VL
    xcls = xcls_ref[...].reshape(nvis, _DV)
    v = _dot(xcls, wv_ref[...]) + bv_ref[...]
    t = _dot(xtxt_ref[0], wt_ref[...]) + bt_ref[...]
    ve = _dot(v, we_ref[...]) + be_ref[...]
    te = _dot(t, we_ref[...]) + be_ref[...]
    x = jnp.concatenate([ve, te], axis=0) * math.sqrt(_DM)
    r = lax.broadcasted_iota(jnp.int32, (_TG, 1), 0)
    is_txt = r >= nvis
    x = x + jnp.where(is_txt, mod0_ref[...], mod3_ref[...])

    rc = lax.broadcasted_iota(jnp.int32, (1, _TG), 1)
    blk_r = jnp.where(r < nvis, r // _VL, r - nvis)
    blk_c = jnp.where(rc < nvis, rc // _VL, rc - nvis)
    bias = jnp.where(blk_r == blk_c, 0.0, -1e30)

    for i in range(_NL):
        (wq, bq, wk, bk, wvv, bvv, wo, bo, w1, b1, w2, b2,
         g1, c1, g2, c2) = lrefs[16 * i:16 * (i + 1)]
        q = _dot(x, wq[...]) + bq[...]
        k = _dot(x, wk[...]) + bk[...]
        vv = _dot(x, wvv[...]) + bvv[...]
        heads = []
        for h in range(_NH):
            sl = slice(h * _DH, (h + 1) * _DH)
            s = _dot_t(q[:, sl], k[:, sl]) * (1.0 / math.sqrt(_DH)) + bias
            e = jnp.exp(s)
            a = e / jnp.sum(e, axis=1, keepdims=True)
            heads.append(_dot(a, vv[:, sl]))
        attn = _dot(jnp.concatenate(heads, axis=1), wo[...]) + bo[...]
        x = _ln(x + attn, g1[...], c1[...])
        ff = _dot(jnp.maximum(_dot(x, w1[...]) + b1[...], 0.0),
                  w2[...]) + b2[...]
        x = _ln(x + ff, g2[...], c2[...])

    w = wl_ref[...]
    vals = jnp.concatenate(
        [_dot_t(w, x[n * _VL:(n + 1) * _VL]) for n in range(_BG)], axis=0)
    val_ref[0] = jnp.concatenate(
        [vals, jnp.zeros((_BG, 128 - _VL), jnp.float32)], axis=1)


def _full(shape):
    return pl.BlockSpec(shape, lambda g: (0,) * len(shape))


def _encode(xcls, xtxt3, *weights):
    return pl.pallas_call(
        _enc_body,
        grid=(_NG,),
        in_specs=[
            pl.BlockSpec((_BG, 1, _VL, _DV), lambda g: (g, 0, 0, 0)),
            pl.BlockSpec((1, _BG, 2048), lambda g: (g, 0, 0)),
        ] + [_full(w.shape) for w in weights],
        out_specs=pl.BlockSpec((1, _BG, 128), lambda g: (g, 0, 0)),
        out_shape=jax.ShapeDtypeStruct((_NG, _BG, 128), jnp.float32),
    )(xcls, xtxt3, *weights)


def _vgather(x, idx):
    dn = lax.GatherDimensionNumbers(offset_dims=(), collapsed_slice_dims=(0,),
                                    start_index_map=(0,))
    return lax.gather(x, idx[:, None], dn, (1,),
                      mode=lax.GatherScatterMode.PROMISE_IN_BOUNDS)


def _bfly(x, op, i):
    for k in (1, 2, 4, 8):
        x = op(x, _vgather(x, jnp.bitwise_xor(i, k)))
    return x


_PAIRS_PW = _B * _NSEL // 32
_ROWS_PW = _PAIRS_PW * _NT
_CHUNKS = (_ROWS_PW + _LANES - 1) // _LANES


def _topk_row(row_v, i):
    chunks = [row_v[pl.ds(c * _LANES, _LANES)] for c in range(_NCH)]
    lanes = [i + c * _LANES for c in range(_NCH)]
    ninf = jnp.float32(-jnp.inf)
    sel = [jnp.zeros((_LANES,), jnp.bool_) for _ in range(_NCH)]
    for _s in range(_NSEL):
        masked = [jnp.where(sel[c], ninf, chunks[c]) for c in range(_NCH)]
        m = _bfly(masked[0], jnp.maximum, i)
        for c in range(1, _NCH):
            m = jnp.maximum(m, _bfly(masked[c], jnp.maximum, i))
        amax = jnp.full((_LANES,), _VL, jnp.int32)
        for c in range(_NCH):
            cand = jnp.where(masked[c] == m, lanes[c], _VL)
            amax = jnp.minimum(amax, _bfly(cand, jnp.minimum, i))
        for c in range(_NCH):
            sel[c] = jnp.logical_or(sel[c], lanes[c] == amax)
    out = jnp.zeros((_LANES,), jnp.int32)
    for s in range(_NSEL):
        mn = jnp.full((_LANES,), _VL, jnp.int32)
        for c in range(_NCH):
            cand = jnp.where(sel[c], lanes[c], _VL)
            mn = jnp.minimum(mn, _bfly(cand, jnp.minimum, i))
        out = jnp.where(i == s, mn, out)
        for c in range(_NCH):
            sel[c] = jnp.logical_and(sel[c], lanes[c] != mn)
    return out


def _select_body(vals_ref, table_ref, out_ref, row_v, j_vm, gidx_vm, sidx_vm,
                 buf_a, buf_b, gsem_a, gsem_b, wsem_a, wsem_b):
    wid = lax.axis_index("s") * 2 + lax.axis_index("c")
    b = lax.shift_right_logical(wid, 1)
    s0 = (wid & 1) * _PAIRS_PW
    i = lax.iota(jnp.int32, _LANES)

    pltpu.sync_copy(vals_ref.at[b], row_v)
    j_vm[...] = _topk_row(row_v, i)

    for c in range(_CHUNKS):
        r = jnp.minimum(i + c * _LANES, _ROWS_PW - 1)
        p = (jnp.where(r >= _NT, 1, 0) + jnp.where(r >= 2 * _NT, 1, 0)
             + jnp.where(r >= 3 * _NT, 1, 0))
        s = s0 + p
        t = r - p * _NT
        j = _vgather(j_vm[...], s)
        gidx_vm[c] = (b * _NT + t) * _VL + j
        sidx_vm[c] = (b * _NT + t) * _NSEL + s

    bufs = (buf_a, buf_b)
    gsems = (gsem_a, gsem_b)
    wsems = (wsem_a, wsem_b)
    gh = [None, None]
    wh = [None, None]
    for c in range(_CHUNKS):
        k = c % 2
        if wh[k] is not None:
            wh[k].wait()
        gh[k] = pltpu.async_copy(table_ref.at[gidx_vm.at[c]], bufs[k],
                                 gsems[k])
        gh[k].wait()
        wh[k] = pltpu.async_copy(bufs[k], out_ref.at[sidx_vm.at[c]], wsems[k])
    for k in range(2):
        if wh[k] is not None:
            wh[k].wait()


@functools.cache
def _sc_select():
    return pl.kernel(
        _select_body,
        out_type=jax.ShapeDtypeStruct((_B * _NT * _NSEL, _DV), jnp.float32),
        mesh=plsc.VectorSubcoreMesh(core_axis_name="c", subcore_axis_name="s"),
        scratch_types=[
            pltpu.VMEM((128,), jnp.float32),
            pltpu.VMEM((_LANES,), jnp.int32),
            pltpu.VMEM((_CHUNKS, _LANES), jnp.int32),
            pltpu.VMEM((_CHUNKS, _LANES), jnp.int32),
            pltpu.VMEM((_LANES, _DV), jnp.float32),
            pltpu.VMEM((_LANES, _DV), jnp.float32),
            pltpu.SemaphoreType.DMA,
            pltpu.SemaphoreType.DMA,
            pltpu.SemaphoreType.DMA,
            pltpu.SemaphoreType.DMA,
        ],
    )


def kernel(x_vis, x_txt, params):
    p = params
    r2 = lambda a: a.reshape(1, -1)

    xvis_t = jnp.transpose(x_vis, (0, 2, 1, 3))
    xtxt3 = x_txt.reshape(_NG, _BG, -1)

    lweights = []
    for l in p["layers"]:
        for kname in _LYR_KEYS:
            a = l[kname]
            lweights.append(a if a.ndim == 2 else r2(a))

    vals = _encode(
        xvis_t, xtxt3,
        p["v_emb"]["W"], r2(p["v_emb"]["b"]),
        p["t_emb"]["W"], r2(p["t_emb"]["b"]),
        p["emb"]["W"], r2(p["emb"]["b"]),
        r2(p["mod_table"][0]), r2(p["mod_table"][3]),
        *lweights,
        p["logits"]["W"].reshape(1, -1),
    ).reshape(_B, 128)

    table = xvis_t.reshape(_B * _NT * _VL, _DV)
    out = _sc_select()(vals, table)
    return jnp.transpose(out.reshape(_B, _NT, _NSEL, _DV), (0, 2, 1, 3))
